# baseline (speedup 1.0000x reference)
import functools

import jax
import jax.numpy as jnp
import numpy as np
from jax import lax
from jax.experimental import pallas as pl
from jax.experimental.pallas import tpu as pltpu
from jax.experimental.pallas import tpu_sc as plsc

F = 26
D = 16
B = 16384
VOCAB = 100000
R = F * VOCAB
BF = B * F
EMBED_OUT = F * D
H1, H2 = 128, 64
BN_EPS = 1e-5

NC, NS = 2, 16
NW = NC * NS
ROWS_W = BF // NW
CHUNK = 1664
NCHUNK = ROWS_W // CHUNK
NPAIR = NCHUNK // 2


@functools.cache
def _make_sc_embed():
    mesh = plsc.VectorSubcoreMesh(core_axis_name="c", subcore_axis_name="s")

    @functools.partial(
        pl.kernel,
        mesh=mesh,
        out_type=jax.ShapeDtypeStruct((BF, D), jnp.float32),
        scratch_types=[
            pltpu.VMEM((CHUNK,), jnp.int32),
            pltpu.VMEM((CHUNK,), jnp.int32),
            pltpu.VMEM((CHUNK, D), jnp.float32),
            pltpu.VMEM((CHUNK, D), jnp.float32),
            pltpu.SemaphoreType.DMA,
            pltpu.SemaphoreType.DMA,
            pltpu.SemaphoreType.DMA,
            pltpu.SemaphoreType.DMA,
        ],
        compiler_params=pltpu.CompilerParams(use_tc_tiling_on_sc=False),
    )
    def _sc_embed(idx_hbm, emb_hbm, out_emb,
                  idx_a, idx_b, rows_a, rows_b, gsem_a, gsem_b,
                  osem_a, osem_b):
        wid = lax.axis_index("s") * NC + lax.axis_index("c")
        base0 = wid * ROWS_W
        idx_v = (idx_a, idx_b)
        rows_v = (rows_a, rows_b)
        gsems = (gsem_a, gsem_b)
        osems = (osem_a, osem_b)

        def prep(c, p):
            pltpu.sync_copy(idx_hbm.at[pl.ds(base0 + c * CHUNK, CHUNK)],
                            idx_v[p])
            pltpu.async_copy(emb_hbm.at[idx_v[p]], rows_v[p], gsems[p])

        def flush(c, p):
            pltpu.make_async_copy(emb_hbm.at[idx_v[p]], rows_v[p],
                                  gsems[p]).wait()
            pltpu.async_copy(
                rows_v[p], out_emb.at[pl.ds(base0 + c * CHUNK, CHUNK)],
                osems[p])

        def drain(c, p):
            pltpu.make_async_copy(
                rows_v[p], out_emb.at[pl.ds(base0 + c * CHUNK, CHUNK)],
                osems[p]).wait()

        prep(0, 0)

        def pair2(i, carry):
            c0 = 2 * i

            @pl.when(i > 0)
            def _():
                drain(c0 - 1, 1)

            prep(c0 + 1, 1)
            flush(c0, 0)

            @pl.when(i + 1 < NPAIR)
            def _():
                drain(c0, 0)
                prep(c0 + 2, 0)

            flush(c0 + 1, 1)
            return carry

        lax.fori_loop(0, NPAIR, pair2, 0)
        drain(NCHUNK - 2, 0)
        drain(NCHUNK - 1, 1)

    return _sc_embed


@functools.cache
def _make_sc_linear():
    mesh = plsc.VectorSubcoreMesh(core_axis_name="c", subcore_axis_name="s")

    @functools.partial(
        pl.kernel,
        mesh=mesh,
        out_type=jax.ShapeDtypeStruct((BF,), jnp.float32),
        scratch_types=[
            pltpu.VMEM((CHUNK,), jnp.int32),
            pltpu.VMEM((CHUNK,), jnp.float32),
            pltpu.SemaphoreType.DMA,
        ],
        compiler_params=pltpu.CompilerParams(use_tc_tiling_on_sc=False),
    )
    def _sc_linear(idx_hbm, lin_hbm, out_lin, idx_v, vals_v, sem):
        wid = lax.axis_index("s") * NC + lax.axis_index("c")
        base0 = wid * ROWS_W

        def body(c, carry):
            base = base0 + c * CHUNK
            pltpu.sync_copy(idx_hbm.at[pl.ds(base, CHUNK)], idx_v)
            pltpu.async_copy(lin_hbm.at[idx_v], vals_v, sem).wait()
            pltpu.sync_copy(vals_v, out_lin.at[pl.ds(base, CHUNK)])
            return carry

        lax.fori_loop(0, NCHUNK, body, 0)

    return _sc_linear


BLK = 1024


def _tc_body(h_ref, lin_ref, w1_ref, b1_ref, w2_ref, b2_ref, smat_ref,
             w3c_ref, out_ref):
    h = h_ref[...]
    lin = lin_ref[...]
    linear = jnp.sum(lin, axis=1)

    hh = jnp.sum(h * h, axis=1)
    s = jnp.dot(h, smat_ref[...], preferred_element_type=jnp.float32)
    fm = 0.5 * (jnp.sum(s * s, axis=1) - hh)

    a1 = jnp.dot(h, w1_ref[...], preferred_element_type=jnp.float32)
    a1 = jnp.maximum(a1 + b1_ref[...], 0.0)
    a2 = jnp.dot(a1, w2_ref[...], preferred_element_type=jnp.float32)
    a2 = jnp.maximum(a2 + b2_ref[...], 0.0)
    mlp = jnp.sum(a2 * w3c_ref[...][:, :H2], axis=1) + w3c_ref[0, H2]
    out_ref[...] = linear + fm + mlp


def _tc_mlp(h, linmat, w1f, b1f, w2f, b2f, smat, w3c):
    grid = (B // BLK,)
    return pl.pallas_call(
        _tc_body,
        grid=grid,
        in_specs=[
            pl.BlockSpec((BLK, EMBED_OUT), lambda i: (i, 0)),
            pl.BlockSpec((BLK, F), lambda i: (i, 0)),
            pl.BlockSpec((EMBED_OUT, H1), lambda i: (0, 0)),
            pl.BlockSpec((1, H1), lambda i: (0, 0)),
            pl.BlockSpec((H1, H2), lambda i: (0, 0)),
            pl.BlockSpec((1, H2), lambda i: (0, 0)),
            pl.BlockSpec((EMBED_OUT, D), lambda i: (0, 0)),
            pl.BlockSpec((1, H2 + 1), lambda i: (0, 0)),
        ],
        out_specs=pl.BlockSpec((BLK,), lambda i: (i,)),
        out_shape=jax.ShapeDtypeStruct((B,), jnp.float32),
    )(h, linmat, w1f, b1f, w2f, b2f, smat, w3c)


def kernel(x, emb_table, lin_w, lin_b, W1, b1, g1, be1, W2, b2, g2, be2,
           W3, b3):
    offsets = (jnp.arange(F, dtype=x.dtype) * VOCAB)[None, :]
    idx = (x + offsets).reshape(-1)
    lin_flat = lin_w.reshape(-1)

    rows = _make_sc_embed()(idx, emb_table)
    lin_vals = _make_sc_linear()(idx, lin_flat)
    h = rows.reshape(B, EMBED_OUT)
    linmat = lin_vals.reshape(B, F)

    bn = 1.0 / np.sqrt(1.0 + BN_EPS)
    w1f = W1 * (bn * g1)[None, :]
    b1f = (b1 * bn * g1 + be1).reshape(1, H1)
    w2f = W2 * (bn * g2)[None, :]
    b2f = (b2 * bn * g2 + be2).reshape(1, H2)
    smat = jnp.tile(jnp.eye(D, dtype=jnp.float32), (F, 1))
    w3c = jnp.concatenate([W3.reshape(1, H2), (lin_b + b3).reshape(1, 1)],
                          axis=1)
    return _tc_mlp(h, linmat, w1f, b1f, w2f, b2f, smat, w3c)
